# baseline (speedup 1.0000x reference)
import functools

import jax
import jax.numpy as jnp
from jax import lax
from jax.experimental import pallas as pl
from jax.experimental.pallas import tpu as pltpu
from jax.experimental.pallas import tpu_sc as plsc

DIM = 64
BATCH = 16384
NC, NS = 2, 16
NW = NC * NS
L = 16
NWIN = 7813
WIN_LO = NWIN // NW
WIN_EXTRA = NWIN % NW
RING = 64

_mesh = plsc.VectorSubcoreMesh(core_axis_name="c", subcore_axis_name="s")


@functools.partial(
    pl.kernel,
    mesh=_mesh,
    out_type=jax.ShapeDtypeStruct((BATCH, DIM), jnp.float32),
    scratch_types=[
        pltpu.VMEM((BATCH,), jnp.int32),
        pltpu.VMEM((BATCH + L,), jnp.int32),
        pltpu.VMEM((BATCH + L,), jnp.int32),
        pltpu.VMEM((8, 8, 128), jnp.float32),
        pltpu.VMEM((8, 8, 128), jnp.float32),
        pltpu.VMEM((RING, DIM), jnp.float32),
        pltpu.SemaphoreType.DMA,
        pltpu.SemaphoreType.DMA,
        pltpu.SemaphoreType.DMA,
        pltpu.SemaphoreType.DMA,
    ],
    compiler_params=pltpu.CompilerParams(needs_layout_passes=False),
)
def _gather_kernel(tabT_hbm, idx_hbm, out_hbm, idx_all, hit_idx, hit_pos,
                   buf0, buf1, ring_v, sem_idx, sem0, sem1, sem_out):
    wid = lax.axis_index("s") * NC + lax.axis_index("c")
    w0 = wid * WIN_LO + jnp.minimum(wid, WIN_EXTRA)
    count = WIN_LO + jnp.where(wid < WIN_EXTRA, 1, 0)

    pltpu.sync_copy(idx_hbm, idx_all)

    lanes = lax.iota(jnp.int32, L)
    w1 = w0 + count

    def prefilter(t, cur):
        v = idx_all[pl.ds(t * L, L)]
        wv = v >> 7
        m = (wv >= w0) & (wv < w1)
        plsc.store_compressed(hit_idx.at[pl.ds(cur, L)], v, mask=m)
        plsc.store_compressed(hit_pos.at[pl.ds(cur, L)], t * L + lanes, mask=m)
        cnt = lax.squeeze(
            lax.slice(plsc.all_reduce_population_count(m), (0,), (1,)), (0,))
        return cur + cnt

    nh = lax.fori_loop(0, BATCH // L, prefilter, jnp.int32(0))
    hit_idx[pl.ds(nh, L)] = jnp.full((L,), -1, jnp.int32)

    bufs = (buf0, buf1)
    sems = (sem0, sem1)

    def fetch(w, buf, sem):
        off = pl.multiple_of(w << 7, 128)
        for r in range(8):
            pltpu.make_async_copy(
                tabT_hbm.at[pl.ds(8 * r, 8), pl.ds(off, 128)],
                buf.at[r], sem,
            ).start()

    def wait_win(buf, sem):
        pltpu.make_async_copy(
            tabT_hbm.at[pl.ds(0, 8), pl.ds(0, 128)], buf, sem
        ).wait()

    def drain_one():
        pltpu.make_async_copy(
            out_hbm.at[pl.ds(0, 1)], ring_v.at[pl.ds(0, 1)], sem_out
        ).wait()

    nvec = (nh + L - 1) >> 4

    def scan_window(k, buf, cnt0):
        w = w0 + k

        def per_vec(t, cnt):
            hv = hit_idx[pl.ds(t * L, L)]
            pv = hit_pos[pl.ds(t * L, L)]
            wv = hv >> 7

            for l in range(L):
                s_w = lax.squeeze(lax.slice(wv, (l,), (l + 1,)), (0,))
                s = lax.squeeze(lax.slice(hv, (l,), (l + 1,)), (0,))
                j = lax.squeeze(lax.slice(pv, (l,), (l + 1,)), (0,))

                def hit_fn(c):
                    @pl.when(c >= RING)
                    def _():
                        drain_one()
                    slot = c & (RING - 1)
                    col = jnp.full((L,), s & 127, jnp.int32)
                    for q in range(4):
                        dvec = 16 * q + lanes
                        vals = plsc.load_gather(
                            buf, [dvec >> 3, dvec & 7, col])
                        ring_v[slot, pl.ds(16 * q, L)] = vals
                    pltpu.make_async_copy(
                        ring_v.at[pl.ds(slot, 1)],
                        out_hbm.at[pl.ds(j, 1)], sem_out,
                    ).start()
                    return c + 1

                cnt = lax.cond(s_w == w, hit_fn, lambda c: c, cnt)
            return cnt

        return lax.fori_loop(0, nvec, per_vec, cnt0)

    fetch(w0, buf0, sem0)

    @pl.when(count > 1)
    def _():
        fetch(w0 + 1, buf1, sem1)

    def outer(kk, cnt):
        for b in range(2):
            k = 2 * kk + b

            def step(cnt):
                wait_win(bufs[b], sems[b])
                cnt2 = scan_window(k, bufs[b], cnt)

                @pl.when(k + 2 < count)
                def _():
                    fetch(w0 + k + 2, bufs[b], sems[b])

                return cnt2

            cnt = lax.cond(k < count, step, lambda c: c, cnt)
        return cnt

    total = lax.fori_loop(0, (count + 1) >> 1, outer, jnp.int32(0))

    def drain(i, _):
        drain_one()
        return ()

    lax.fori_loop(0, jnp.minimum(total, RING), drain, ())


def kernel(inputs, table):
    return _gather_kernel(table.T, inputs)


# speedup vs baseline: 10.4099x; 10.4099x over previous
import functools

import jax
import jax.numpy as jnp
from jax import lax
from jax.experimental import pallas as pl
from jax.experimental.pallas import tpu as pltpu
from jax.experimental.pallas import tpu_sc as plsc

DIM = 64
BATCH = 16384
NC, NS = 2, 16
NW = NC * NS
L = 16
NTILECOL = 7813
WINC = 4
NWIN = 1954
WIN_LO = NWIN // NW
WIN_EXTRA = NWIN % NW
RING = 64

_mesh = plsc.VectorSubcoreMesh(core_axis_name="c", subcore_axis_name="s")


@functools.partial(
    pl.kernel,
    mesh=_mesh,
    out_type=jax.ShapeDtypeStruct((BATCH, DIM), jnp.float32),
    scratch_types=[
        pltpu.VMEM((BATCH,), jnp.int32),
        pltpu.VMEM((BATCH + L,), jnp.int32),
        pltpu.VMEM((BATCH + L,), jnp.int32),
        pltpu.VMEM((WINC, 8, 8, 128), jnp.float32),
        pltpu.VMEM((WINC, 8, 8, 128), jnp.float32),
        pltpu.VMEM((RING, DIM), jnp.float32),
        pltpu.SemaphoreType.DMA,
        pltpu.SemaphoreType.DMA,
        pltpu.SemaphoreType.DMA,
    ],
    compiler_params=pltpu.CompilerParams(needs_layout_passes=False),
)
def _gather_kernel(tabT_hbm, idx_hbm, out_hbm, idx_all, hit_idx, hit_pos,
                   buf0, buf1, ring_v, sem0, sem1, sem_out):
    wid = lax.axis_index("s") * NC + lax.axis_index("c")
    w0 = wid * WIN_LO + jnp.minimum(wid, WIN_EXTRA)
    count = WIN_LO + jnp.where(wid < WIN_EXTRA, 1, 0)

    pltpu.sync_copy(idx_hbm, idx_all)

    lanes = lax.iota(jnp.int32, L)
    w1 = w0 + count

    def prefilter(t, cur):
        v = idx_all[pl.ds(t * L, L)]
        wv = v >> 9
        m = (wv >= w0) & (wv < w1)
        plsc.store_compressed(hit_idx.at[pl.ds(cur, L)], v, mask=m)
        plsc.store_compressed(hit_pos.at[pl.ds(cur, L)], t * L + lanes, mask=m)
        cnt = lax.squeeze(
            lax.slice(plsc.all_reduce_population_count(m), (0,), (1,)), (0,))
        return cur + cnt

    nh = lax.fori_loop(0, BATCH // L, prefilter, jnp.int32(0))
    hit_idx[pl.ds(nh, L)] = jnp.full((L,), -1, jnp.int32)

    bufs = (buf0, buf1)
    sems = (sem0, sem1)

    def fetch(w, buf, sem):
        for kc in range(WINC):
            tc = w * WINC + kc
            off = pl.multiple_of(jnp.where(tc < NTILECOL, tc, 0) << 7, 128)
            for r in range(8):
                pltpu.make_async_copy(
                    tabT_hbm.at[pl.ds(8 * r, 8), pl.ds(off, 128)],
                    buf.at[kc, r], sem,
                ).start()

    def wait_win(buf, sem):
        pltpu.make_async_copy(
            tabT_hbm.at[pl.ds(0, 8), pl.ds(0, 128)], buf, sem
        ).wait()

    def drain_one():
        pltpu.make_async_copy(
            out_hbm.at[pl.ds(0, 1)], ring_v.at[pl.ds(0, 1)], sem_out
        ).wait()

    nvec = (nh + L - 1) >> 4

    def scan_window(k, buf, cnt0):
        w = w0 + k

        def per_vec(t, cnt):
            hv = hit_idx[pl.ds(t * L, L)]
            wv = hv >> 9
            nmatch = lax.squeeze(
                lax.slice(plsc.all_reduce_population_count(wv == w),
                          (0,), (1,)), (0,))

            def vec_hits(c_in):
                pv = hit_pos[pl.ds(t * L, L)]
                c_out = c_in
                for l in range(L):
                    s_w = lax.squeeze(lax.slice(wv, (l,), (l + 1,)), (0,))
                    s = lax.squeeze(lax.slice(hv, (l,), (l + 1,)), (0,))
                    j = lax.squeeze(lax.slice(pv, (l,), (l + 1,)), (0,))

                    def hit_fn(c):
                        @pl.when(c >= RING)
                        def _():
                            drain_one()
                        slot = c & (RING - 1)
                        l9 = s & 511
                        kcol = jnp.full((L,), l9 >> 7, jnp.int32)
                        col = jnp.full((L,), l9 & 127, jnp.int32)
                        for q in range(4):
                            dvec = 16 * q + lanes
                            vals = plsc.load_gather(
                                buf, [kcol, dvec >> 3, dvec & 7, col])
                            ring_v[slot, pl.ds(16 * q, L)] = vals
                        pltpu.make_async_copy(
                            ring_v.at[pl.ds(slot, 1)],
                            out_hbm.at[pl.ds(j, 1)], sem_out,
                        ).start()
                        return c + 1

                    c_out = lax.cond(s_w == w, hit_fn, lambda c: c, c_out)
                return c_out

            return lax.cond(nmatch > 0, vec_hits, lambda c: c, cnt)

        return lax.fori_loop(0, nvec, per_vec, cnt0)

    fetch(w0, buf0, sem0)

    @pl.when(count > 1)
    def _():
        fetch(w0 + 1, buf1, sem1)

    def outer(kk, cnt):
        for b in range(2):
            k = 2 * kk + b

            def step(cnt):
                wait_win(bufs[b], sems[b])
                cnt2 = scan_window(k, bufs[b], cnt)

                @pl.when(k + 2 < count)
                def _():
                    fetch(w0 + k + 2, bufs[b], sems[b])

                return cnt2

            cnt = lax.cond(k < count, step, lambda c: c, cnt)
        return cnt

    total = lax.fori_loop(0, (count + 1) >> 1, outer, jnp.int32(0))

    def drain(i, _):
        drain_one()
        return ()

    lax.fori_loop(0, jnp.minimum(total, RING), drain, ())


def kernel(inputs, table):
    return _gather_kernel(table.T, inputs)


# speedup vs baseline: 17.7573x; 1.7058x over previous
import functools

import jax
import jax.numpy as jnp
from jax import lax
from jax.experimental import pallas as pl
from jax.experimental.pallas import tpu as pltpu
from jax.experimental.pallas import tpu_sc as plsc

DIM = 64
BATCH = 16384
NC, NS = 2, 16
NW = NC * NS
L = 16
NTILECOL = 7813
WINC = 4
NWIN = 1954
WIN_LO = NWIN // NW
WIN_EXTRA = NWIN % NW
WMAX = WIN_LO + 1
RING = 64

_mesh = plsc.VectorSubcoreMesh(core_axis_name="c", subcore_axis_name="s")


@functools.partial(
    pl.kernel,
    mesh=_mesh,
    out_type=jax.ShapeDtypeStruct((BATCH, DIM), jnp.float32),
    scratch_types=[
        pltpu.VMEM((BATCH + L,), jnp.int32),
        pltpu.VMEM((BATCH + L,), jnp.int32),
        pltpu.VMEM((BATCH + L,), jnp.int32),
        pltpu.VMEM((WINC, 8, 8, 128), jnp.float32),
        pltpu.VMEM((WINC, 8, 8, 128), jnp.float32),
        pltpu.VMEM((RING, DIM), jnp.float32),
        pltpu.SMEM((WMAX + 1,), jnp.int32),
        pltpu.SMEM((WMAX + 1,), jnp.int32),
        pltpu.SemaphoreType.DMA,
        pltpu.SemaphoreType.DMA,
        pltpu.SemaphoreType.DMA,
    ],
    compiler_params=pltpu.CompilerParams(needs_layout_passes=False),
)
def _gather_kernel(tabT_hbm, idx_hbm, out_hbm, sorted_v, hit_win, hit_val,
                   buf0, buf1, ring_v, starts_s, cur_s,
                   sem0, sem1, sem_out):
    wid = lax.axis_index("s") * NC + lax.axis_index("c")
    w0 = wid * WIN_LO + jnp.minimum(wid, WIN_EXTRA)
    count = WIN_LO + jnp.where(wid < WIN_EXTRA, 1, 0)

    pltpu.sync_copy(idx_hbm, sorted_v.at[pl.ds(0, BATCH)])

    lanes = lax.iota(jnp.int32, L)
    w1 = w0 + count

    def prefilter(t, cur):
        v = sorted_v[pl.ds(t * L, L)]
        wv = v >> 9
        m = (wv >= w0) & (wv < w1)
        plsc.store_compressed(hit_win.at[pl.ds(cur, L)], wv - w0, mask=m)
        plsc.store_compressed(
            hit_val.at[pl.ds(cur, L)],
            ((t * L + lanes) << 9) | (v & 511), mask=m)
        cnt = lax.squeeze(
            lax.slice(plsc.all_reduce_population_count(m), (0,), (1,)), (0,))
        return cur + cnt

    nh = lax.fori_loop(0, BATCH // L, prefilter, jnp.int32(0))
    hit_win[pl.ds(nh, L)] = jnp.full((L,), -1, jnp.int32)
    nvec = (nh + L - 1) >> 4

    def zero(k, _):
        cur_s[k] = 0
        return ()

    lax.fori_loop(0, WMAX + 1, zero, ())

    def count_pass(t, _):
        wv = hit_win[pl.ds(t * L, L)]
        for l in range(L):
            w = lax.squeeze(lax.slice(wv, (l,), (l + 1,)), (0,))

            @pl.when(w >= 0)
            def _():
                cur_s[w] = cur_s[w] + 1
            return_val = None
        return ()

    lax.fori_loop(0, nvec, count_pass, ())

    starts_s[0] = 0

    def prefix(k, _):
        starts_s[k + 1] = starts_s[k] + cur_s[k]
        cur_s[k] = starts_s[k]
        return ()

    lax.fori_loop(0, WMAX, prefix, ())

    def place_pass(t, _):
        wv = hit_win[pl.ds(t * L, L)]
        vv = hit_val[pl.ds(t * L, L)]
        for l in range(L):
            w = lax.squeeze(lax.slice(wv, (l,), (l + 1,)), (0,))

            @pl.when(w >= 0)
            def _():
                d = cur_s[w]
                cur_s[w] = d + 1
                plsc.store_scatter(
                    sorted_v, [jnp.full((L,), d, jnp.int32)], vv,
                    mask=lanes == l)
        return ()

    lax.fori_loop(0, nvec, place_pass, ())

    bufs = (buf0, buf1)
    sems = (sem0, sem1)

    def fetch(w, buf, sem):
        for kc in range(WINC):
            tc = w * WINC + kc
            off = pl.multiple_of(jnp.where(tc < NTILECOL, tc, 0) << 7, 128)
            for r in range(8):
                pltpu.make_async_copy(
                    tabT_hbm.at[pl.ds(8 * r, 8), pl.ds(off, 128)],
                    buf.at[kc, r], sem,
                ).start()

    def wait_win(buf, sem):
        pltpu.make_async_copy(
            tabT_hbm.at[pl.ds(0, 8), pl.ds(0, 128)], buf, sem
        ).wait()

    def drain_one():
        pltpu.make_async_copy(
            out_hbm.at[pl.ds(0, 1)], ring_v.at[pl.ds(0, 1)], sem_out
        ).wait()

    def scan_window(k, buf, cnt0):
        s0 = starts_s[k]
        s1 = starts_s[k + 1]

        def per_hit(h, c):
            val = lax.squeeze(
                lax.slice(sorted_v[pl.ds(h, L)], (0,), (1,)), (0,))

            @pl.when(c >= RING)
            def _():
                drain_one()
            slot = c & (RING - 1)
            l9 = val & 511
            j = val >> 9
            kcol = jnp.full((L,), l9 >> 7, jnp.int32)
            col = jnp.full((L,), l9 & 127, jnp.int32)
            for q in range(4):
                dvec = 16 * q + lanes
                vals = plsc.load_gather(buf, [kcol, dvec >> 3, dvec & 7, col])
                ring_v[slot, pl.ds(16 * q, L)] = vals
            pltpu.make_async_copy(
                ring_v.at[pl.ds(slot, 1)],
                out_hbm.at[pl.ds(j, 1)], sem_out,
            ).start()
            return c + 1

        return lax.fori_loop(s0, s1, per_hit, cnt0)

    fetch(w0, buf0, sem0)

    @pl.when(count > 1)
    def _():
        fetch(w0 + 1, buf1, sem1)

    def outer(kk, cnt):
        for b in range(2):
            k = 2 * kk + b

            def step(cnt):
                wait_win(bufs[b], sems[b])
                cnt2 = scan_window(k, bufs[b], cnt)

                @pl.when(k + 2 < count)
                def _():
                    fetch(w0 + k + 2, bufs[b], sems[b])

                return cnt2

            cnt = lax.cond(k < count, step, lambda c: c, cnt)
        return cnt

    total = lax.fori_loop(0, (count + 1) >> 1, outer, jnp.int32(0))

    def drain(i, _):
        drain_one()
        return ()

    lax.fori_loop(0, jnp.minimum(total, RING), drain, ())


def kernel(inputs, table):
    return _gather_kernel(table.T, inputs)


# speedup vs baseline: 18.1800x; 1.0238x over previous
import functools

import jax
import jax.numpy as jnp
from jax import lax
from jax.experimental import pallas as pl
from jax.experimental.pallas import tpu as pltpu
from jax.experimental.pallas import tpu_sc as plsc

DIM = 64
BATCH = 16384
NC, NS = 2, 16
NW = NC * NS
L = 16
NTILECOL = 7813
WINC = 4
NWIN = 1954
WIN_LO = NWIN // NW
WIN_EXTRA = NWIN % NW
WMAX = WIN_LO + 1
RING = 64

_mesh = plsc.VectorSubcoreMesh(core_axis_name="c", subcore_axis_name="s")


@functools.partial(
    pl.kernel,
    mesh=_mesh,
    out_type=jax.ShapeDtypeStruct((BATCH, DIM), jnp.float32),
    scratch_types=[
        pltpu.VMEM((BATCH + L,), jnp.int32),
        pltpu.VMEM((BATCH + L,), jnp.int32),
        pltpu.VMEM((BATCH + L,), jnp.int32),
        pltpu.VMEM((WINC, 8, 8, 128), jnp.float32),
        pltpu.VMEM((WINC, 8, 8, 128), jnp.float32),
        pltpu.VMEM((RING, DIM), jnp.float32),
        pltpu.SMEM((WMAX + 1,), jnp.int32),
        pltpu.SMEM((WMAX + 1,), jnp.int32),
        pltpu.SemaphoreType.DMA,
        pltpu.SemaphoreType.DMA,
        pltpu.SemaphoreType.DMA,
    ],
    compiler_params=pltpu.CompilerParams(needs_layout_passes=False),
)
def _gather_kernel(tabT_hbm, idx_hbm, out_hbm, sorted_v, hit_win, hit_val,
                   buf0, buf1, ring_v, starts_s, cur_s,
                   sem0, sem1, sem_out):
    wid = lax.axis_index("s") * NC + lax.axis_index("c")
    w0 = wid * WIN_LO + jnp.minimum(wid, WIN_EXTRA)
    count = WIN_LO + jnp.where(wid < WIN_EXTRA, 1, 0)

    pltpu.sync_copy(idx_hbm, sorted_v.at[pl.ds(0, BATCH)])

    lanes = lax.iota(jnp.int32, L)
    w1 = w0 + count

    def fetch(w, buf, sem):
        for kc in range(WINC):
            tc = w * WINC + kc
            off = pl.multiple_of(jnp.where(tc < NTILECOL, tc, 0) << 7, 128)
            for r in range(8):
                pltpu.make_async_copy(
                    tabT_hbm.at[pl.ds(8 * r, 8), pl.ds(off, 128)],
                    buf.at[kc, r], sem,
                ).start()

    fetch(w0, buf0, sem0)

    @pl.when(count > 1)
    def _():
        fetch(w0 + 1, buf1, sem1)

    def prefilter(t, cur):
        v = sorted_v[pl.ds(t * L, L)]
        wv = v >> 9
        m = (wv >= w0) & (wv < w1)
        plsc.store_compressed(hit_win.at[pl.ds(cur, L)], wv - w0, mask=m)
        plsc.store_compressed(
            hit_val.at[pl.ds(cur, L)],
            ((t * L + lanes) << 9) | (v & 511), mask=m)
        cnt = lax.squeeze(
            lax.slice(plsc.all_reduce_population_count(m), (0,), (1,)), (0,))
        return cur + cnt

    nh = lax.fori_loop(0, BATCH // L, prefilter, jnp.int32(0))
    hit_win[pl.ds(nh, L)] = jnp.full((L,), -1, jnp.int32)
    nvec = (nh + L - 1) >> 4

    def zero(k, _):
        cur_s[k] = 0
        return ()

    lax.fori_loop(0, WMAX + 1, zero, ())

    def count_pass(t, _):
        wv = hit_win[pl.ds(t * L, L)]
        for l in range(L):
            w = lax.squeeze(lax.slice(wv, (l,), (l + 1,)), (0,))

            @pl.when(w >= 0)
            def _():
                cur_s[w] = cur_s[w] + 1
            return_val = None
        return ()

    lax.fori_loop(0, nvec, count_pass, ())

    starts_s[0] = 0

    def prefix(k, _):
        starts_s[k + 1] = starts_s[k] + cur_s[k]
        cur_s[k] = starts_s[k]
        return ()

    lax.fori_loop(0, WMAX, prefix, ())

    def place_pass(t, _):
        wv = hit_win[pl.ds(t * L, L)]
        vv = hit_val[pl.ds(t * L, L)]
        for l in range(L):
            w = lax.squeeze(lax.slice(wv, (l,), (l + 1,)), (0,))

            @pl.when(w >= 0)
            def _():
                d = cur_s[w]
                cur_s[w] = d + 1
                plsc.store_scatter(
                    sorted_v, [jnp.full((L,), d, jnp.int32)], vv,
                    mask=lanes == l)
        return ()

    lax.fori_loop(0, nvec, place_pass, ())

    bufs = (buf0, buf1)
    sems = (sem0, sem1)

    def wait_win(buf, sem):
        pltpu.make_async_copy(
            tabT_hbm.at[pl.ds(0, 8), pl.ds(0, 128)], buf, sem
        ).wait()

    def drain_one():
        pltpu.make_async_copy(
            out_hbm.at[pl.ds(0, 1)], ring_v.at[pl.ds(0, 1)], sem_out
        ).wait()

    def scan_window(k, buf, cnt0):
        s0 = starts_s[k]
        s1 = starts_s[k + 1]

        def per_hit(h, c):
            val = lax.squeeze(
                lax.slice(sorted_v[pl.ds(h, L)], (0,), (1,)), (0,))

            @pl.when(c >= RING)
            def _():
                drain_one()
            slot = c & (RING - 1)
            l9 = val & 511
            j = val >> 9
            kcol = jnp.full((L,), l9 >> 7, jnp.int32)
            col = jnp.full((L,), l9 & 127, jnp.int32)
            for q in range(4):
                dvec = 16 * q + lanes
                vals = plsc.load_gather(buf, [kcol, dvec >> 3, dvec & 7, col])
                ring_v[slot, pl.ds(16 * q, L)] = vals
            pltpu.make_async_copy(
                ring_v.at[pl.ds(slot, 1)],
                out_hbm.at[pl.ds(j, 1)], sem_out,
            ).start()
            return c + 1

        return lax.fori_loop(s0, s1, per_hit, cnt0)

    def outer(kk, cnt):
        for b in range(2):
            k = 2 * kk + b

            def step(cnt):
                wait_win(bufs[b], sems[b])
                cnt2 = scan_window(k, bufs[b], cnt)

                @pl.when(k + 2 < count)
                def _():
                    fetch(w0 + k + 2, bufs[b], sems[b])

                return cnt2

            cnt = lax.cond(k < count, step, lambda c: c, cnt)
        return cnt

    total = lax.fori_loop(0, (count + 1) >> 1, outer, jnp.int32(0))

    def drain(i, _):
        drain_one()
        return ()

    lax.fori_loop(0, jnp.minimum(total, RING), drain, ())


def kernel(inputs, table):
    return _gather_kernel(table.T, inputs)


# speedup vs baseline: 18.2893x; 1.0060x over previous
import functools

import jax
import jax.numpy as jnp
from jax import lax
from jax.experimental import pallas as pl
from jax.experimental.pallas import tpu as pltpu
from jax.experimental.pallas import tpu_sc as plsc

DIM = 64
BATCH = 16384
NC, NS = 2, 16
NW = NC * NS
L = 16
NTILECOL = 7813
WINC = 4
NWIN = 1954
WIN_LO = NWIN // NW
WIN_EXTRA = NWIN % NW
WMAX = WIN_LO + 1
RING = 64

_mesh = plsc.VectorSubcoreMesh(core_axis_name="c", subcore_axis_name="s")


@functools.partial(
    pl.kernel,
    mesh=_mesh,
    out_type=jax.ShapeDtypeStruct((BATCH, DIM), jnp.float32),
    scratch_types=[
        pltpu.VMEM((BATCH + L,), jnp.int32),
        pltpu.VMEM((BATCH + L,), jnp.int32),
        pltpu.VMEM((BATCH + L,), jnp.int32),
        pltpu.VMEM((WINC, 64, 128), jnp.float32),
        pltpu.VMEM((WINC, 64, 128), jnp.float32),
        pltpu.VMEM((RING, DIM), jnp.float32),
        pltpu.SMEM((WMAX + 1,), jnp.int32),
        pltpu.SMEM((WMAX + 1,), jnp.int32),
        pltpu.SemaphoreType.DMA,
        pltpu.SemaphoreType.DMA,
        pltpu.SemaphoreType.DMA,
    ],
    compiler_params=pltpu.CompilerParams(needs_layout_passes=False),
)
def _gather_kernel(tabT_hbm, idx_hbm, out_hbm, sorted_v, hit_win, hit_val,
                   buf0, buf1, ring_v, starts_s, cur_s,
                   sem0, sem1, sem_out):
    wid = lax.axis_index("s") * NC + lax.axis_index("c")
    w0 = wid * WIN_LO + jnp.minimum(wid, WIN_EXTRA)
    count = WIN_LO + jnp.where(wid < WIN_EXTRA, 1, 0)

    pltpu.sync_copy(idx_hbm, sorted_v.at[pl.ds(0, BATCH)])

    lanes = lax.iota(jnp.int32, L)
    w1 = w0 + count

    def fetch(w, buf, sem):
        for kc in range(WINC):
            tc = w * WINC + kc
            off = pl.multiple_of(jnp.where(tc < NTILECOL, tc, 0) << 7, 128)
            pltpu.make_async_copy(
                tabT_hbm.at[pl.ds(0, 64), pl.ds(off, 128)],
                buf.at[kc], sem,
            ).start()

    fetch(w0, buf0, sem0)

    @pl.when(count > 1)
    def _():
        fetch(w0 + 1, buf1, sem1)

    def prefilter(t, cur):
        v = sorted_v[pl.ds(t * L, L)]
        wv = v >> 9
        m = (wv >= w0) & (wv < w1)
        plsc.store_compressed(hit_win.at[pl.ds(cur, L)], wv - w0, mask=m)
        plsc.store_compressed(
            hit_val.at[pl.ds(cur, L)],
            ((t * L + lanes) << 9) | (v & 511), mask=m)
        cnt = lax.squeeze(
            lax.slice(plsc.all_reduce_population_count(m), (0,), (1,)), (0,))
        return cur + cnt

    nh = lax.fori_loop(0, BATCH // L, prefilter, jnp.int32(0))
    hit_win[pl.ds(nh, L)] = jnp.full((L,), -1, jnp.int32)
    nvec = (nh + L - 1) >> 4

    def zero(k, _):
        cur_s[k] = 0
        return ()

    lax.fori_loop(0, WMAX + 1, zero, ())

    def count_pass(t, _):
        wv = hit_win[pl.ds(t * L, L)]
        for l in range(L):
            w = lax.squeeze(lax.slice(wv, (l,), (l + 1,)), (0,))

            @pl.when(w >= 0)
            def _():
                cur_s[w] = cur_s[w] + 1
            return_val = None
        return ()

    lax.fori_loop(0, nvec, count_pass, ())

    starts_s[0] = 0

    def prefix(k, _):
        starts_s[k + 1] = starts_s[k] + cur_s[k]
        cur_s[k] = starts_s[k]
        return ()

    lax.fori_loop(0, WMAX, prefix, ())

    def place_pass(t, _):
        wv = hit_win[pl.ds(t * L, L)]
        vv = hit_val[pl.ds(t * L, L)]
        for l in range(L):
            w = lax.squeeze(lax.slice(wv, (l,), (l + 1,)), (0,))

            @pl.when(w >= 0)
            def _():
                d = cur_s[w]
                cur_s[w] = d + 1
                plsc.store_scatter(
                    sorted_v, [jnp.full((L,), d, jnp.int32)], vv,
                    mask=lanes == l)
        return ()

    lax.fori_loop(0, nvec, place_pass, ())

    bufs = (buf0, buf1)
    sems = (sem0, sem1)

    def wait_win(buf, sem):
        pltpu.make_async_copy(
            tabT_hbm.at[pl.ds(0, 8), pl.ds(0, 128)], buf, sem
        ).wait()

    def drain_one():
        pltpu.make_async_copy(
            out_hbm.at[pl.ds(0, 1)], ring_v.at[pl.ds(0, 1)], sem_out
        ).wait()

    def scan_window(k, buf, cnt0):
        s0 = starts_s[k]
        s1 = starts_s[k + 1]

        def per_hit(h, c):
            val = lax.squeeze(
                lax.slice(sorted_v[pl.ds(h, L)], (0,), (1,)), (0,))

            @pl.when(c >= RING)
            def _():
                drain_one()
            slot = c & (RING - 1)
            l9 = val & 511
            j = val >> 9
            kcol = jnp.full((L,), l9 >> 7, jnp.int32)
            col = jnp.full((L,), l9 & 127, jnp.int32)
            for q in range(4):
                dvec = 16 * q + lanes
                vals = plsc.load_gather(buf, [kcol, dvec, col])
                ring_v[slot, pl.ds(16 * q, L)] = vals
            pltpu.make_async_copy(
                ring_v.at[pl.ds(slot, 1)],
                out_hbm.at[pl.ds(j, 1)], sem_out,
            ).start()
            return c + 1

        return lax.fori_loop(s0, s1, per_hit, cnt0)

    def outer(kk, cnt):
        for b in range(2):
            k = 2 * kk + b

            def step(cnt):
                wait_win(bufs[b], sems[b])
                cnt2 = scan_window(k, bufs[b], cnt)

                @pl.when(k + 2 < count)
                def _():
                    fetch(w0 + k + 2, bufs[b], sems[b])

                return cnt2

            cnt = lax.cond(k < count, step, lambda c: c, cnt)
        return cnt

    total = lax.fori_loop(0, (count + 1) >> 1, outer, jnp.int32(0))

    def drain(i, _):
        drain_one()
        return ()

    lax.fori_loop(0, jnp.minimum(total, RING), drain, ())


def kernel(inputs, table):
    return _gather_kernel(table.T, inputs)


# speedup vs baseline: 20.6504x; 1.1291x over previous
import functools

import jax
import jax.numpy as jnp
from jax import lax
from jax.experimental import pallas as pl
from jax.experimental.pallas import tpu as pltpu
from jax.experimental.pallas import tpu_sc as plsc

DIM = 64
BATCH = 16384
NC, NS = 2, 16
NW = NC * NS
L = 16
NTILECOL = 7813
WINC = 4
NWIN = 1954
WIN_LO = NWIN // NW
WIN_EXTRA = NWIN % NW
WMAX = WIN_LO + 1
CMAX = 4 * WMAX
RING = 64

_mesh = plsc.VectorSubcoreMesh(core_axis_name="c", subcore_axis_name="s")


@functools.partial(
    pl.kernel,
    mesh=_mesh,
    out_type=jax.ShapeDtypeStruct((BATCH, DIM), jnp.float32),
    scratch_types=[
        pltpu.VMEM((BATCH + L,), jnp.int32),
        pltpu.VMEM((BATCH + L,), jnp.int32),
        pltpu.VMEM((BATCH + L,), jnp.int32),
        pltpu.VMEM((WINC, 64, 128), jnp.float32),
        pltpu.VMEM((WINC, 64, 128), jnp.float32),
        pltpu.VMEM((RING, DIM), jnp.float32),
        pltpu.SMEM((CMAX + 1,), jnp.int32),
        pltpu.SMEM((CMAX + 1,), jnp.int32),
        pltpu.SemaphoreType.DMA,
        pltpu.SemaphoreType.DMA,
        pltpu.SemaphoreType.DMA,
    ],
    compiler_params=pltpu.CompilerParams(needs_layout_passes=False),
)
def _gather_kernel(tabT_hbm, idx_hbm, out_hbm, sorted_v, hit_win, hit_val,
                   buf0, buf1, ring_v, starts_s, cur_s,
                   sem0, sem1, sem_out):
    wid = lax.axis_index("s") * NC + lax.axis_index("c")
    w0 = wid * WIN_LO + jnp.minimum(wid, WIN_EXTRA)
    count = WIN_LO + jnp.where(wid < WIN_EXTRA, 1, 0)

    pltpu.sync_copy(idx_hbm, sorted_v.at[pl.ds(0, BATCH)])

    lanes = lax.iota(jnp.int32, L)
    w1 = w0 + count

    def fetch_col(w, kc, buf, sem):
        tc = w * WINC + kc
        off = pl.multiple_of(jnp.where(tc < NTILECOL, tc, 0) << 7, 128)
        pltpu.make_async_copy(
            tabT_hbm.at[pl.ds(0, 64), pl.ds(off, 128)],
            buf.at[kc], sem,
        ).start()

    def fetch(w, buf, sem):
        for kc in range(WINC):
            fetch_col(w, kc, buf, sem)

    fetch(w0, buf0, sem0)

    @pl.when(count > 1)
    def _():
        fetch(w0 + 1, buf1, sem1)

    def prefilter(t, cur):
        v = sorted_v[pl.ds(t * L, L)]
        wv = v >> 9
        m = (wv >= w0) & (wv < w1)
        plsc.store_compressed(
            hit_win.at[pl.ds(cur, L)], (v >> 7) - 4 * w0, mask=m)
        plsc.store_compressed(
            hit_val.at[pl.ds(cur, L)],
            ((t * L + lanes) << 7) | (v & 127), mask=m)
        cnt = lax.squeeze(
            lax.slice(plsc.all_reduce_population_count(m), (0,), (1,)), (0,))
        return cur + cnt

    nh = lax.fori_loop(0, BATCH // L, prefilter, jnp.int32(0))
    hit_win[pl.ds(nh, L)] = jnp.full((L,), -1, jnp.int32)
    nvec = (nh + L - 1) >> 4

    def zero(k, _):
        cur_s[k] = 0
        return ()

    lax.fori_loop(0, CMAX + 1, zero, ())

    def count_pass(t, _):
        wv = hit_win[pl.ds(t * L, L)]
        for l in range(L):
            w = lax.squeeze(lax.slice(wv, (l,), (l + 1,)), (0,))

            @pl.when(w >= 0)
            def _():
                cur_s[w] = cur_s[w] + 1
            return_val = None
        return ()

    lax.fori_loop(0, nvec, count_pass, ())

    starts_s[0] = 0

    def prefix(k, _):
        starts_s[k + 1] = starts_s[k] + cur_s[k]
        cur_s[k] = starts_s[k]
        return ()

    lax.fori_loop(0, CMAX, prefix, ())

    def place_pass(t, _):
        wv = hit_win[pl.ds(t * L, L)]
        vv = hit_val[pl.ds(t * L, L)]
        for l in range(L):
            w = lax.squeeze(lax.slice(wv, (l,), (l + 1,)), (0,))

            @pl.when(w >= 0)
            def _():
                d = cur_s[w]
                cur_s[w] = d + 1
                plsc.store_scatter(
                    sorted_v, [jnp.full((L,), d, jnp.int32)], vv,
                    mask=lanes == l)
        return ()

    lax.fori_loop(0, nvec, place_pass, ())

    bufs = (buf0, buf1)
    sems = (sem0, sem1)

    def wait_col(buf, kc, sem):
        pltpu.make_async_copy(
            tabT_hbm.at[pl.ds(0, 64), pl.ds(0, 128)], buf.at[kc], sem
        ).wait()

    def drain_one():
        pltpu.make_async_copy(
            out_hbm.at[pl.ds(0, 1)], ring_v.at[pl.ds(0, 1)], sem_out
        ).wait()

    def scan_col(k, kc, buf, cnt0):
        c0 = k * WINC + kc
        s0 = starts_s[c0]
        s1 = starts_s[c0 + 1]

        def per_hit(h, c):
            val = lax.squeeze(
                lax.slice(sorted_v[pl.ds(h, L)], (0,), (1,)), (0,))

            @pl.when(c >= RING)
            def _():
                drain_one()
            slot = c & (RING - 1)
            j = val >> 7
            kcol = jnp.full((L,), kc, jnp.int32)
            col = jnp.full((L,), val & 127, jnp.int32)
            for q in range(4):
                dvec = 16 * q + lanes
                vals = plsc.load_gather(buf, [kcol, dvec, col])
                ring_v[slot, pl.ds(16 * q, L)] = vals
            pltpu.make_async_copy(
                ring_v.at[pl.ds(slot, 1)],
                out_hbm.at[pl.ds(j, 1)], sem_out,
            ).start()
            return c + 1

        return lax.fori_loop(s0, s1, per_hit, cnt0)

    def outer(kk, cnt):
        for b in range(2):
            k = 2 * kk + b

            def step(cnt):
                cnt2 = cnt
                for kc in range(WINC):
                    c0 = k * WINC + kc
                    used = (k < 2) | (starts_s[c0 + 1] > starts_s[c0])

                    @pl.when(used)
                    def _():
                        wait_col(bufs[b], kc, sems[b])
                    cnt2 = lax.cond(
                        starts_s[c0 + 1] > starts_s[c0],
                        lambda c, kc=kc: scan_col(k, kc, bufs[b], c),
                        lambda c: c, cnt2)

                @pl.when(k + 2 < count)
                def _():
                    kf = k + 2
                    for kc in range(WINC):
                        cf = kf * WINC + kc

                        @pl.when(starts_s[cf + 1] > starts_s[cf])
                        def _(kc=kc):
                            fetch_col(w0 + kf, kc, bufs[b], sems[b])

                return cnt2

            cnt = lax.cond(k < count, step, lambda c: c, cnt)
        return cnt

    total = lax.fori_loop(0, (count + 1) >> 1, outer, jnp.int32(0))

    def drain(i, _):
        drain_one()
        return ()

    lax.fori_loop(0, jnp.minimum(total, RING), drain, ())


def kernel(inputs, table):
    return _gather_kernel(table.T, inputs)


# speedup vs baseline: 20.6965x; 1.0022x over previous
import functools

import jax
import jax.numpy as jnp
from jax import lax
from jax.experimental import pallas as pl
from jax.experimental.pallas import tpu as pltpu
from jax.experimental.pallas import tpu_sc as plsc

DIM = 64
BATCH = 16384
NC, NS = 2, 16
NW = NC * NS
L = 16
NTILECOL = 7813
WINC = 4
NWIN = 1954
WIN_LO = NWIN // NW
WIN_EXTRA = NWIN % NW
WMAX = WIN_LO + 1
CMAX = 4 * WMAX
RING = 64

_mesh = plsc.VectorSubcoreMesh(core_axis_name="c", subcore_axis_name="s")


@functools.partial(
    pl.kernel,
    mesh=_mesh,
    out_type=jax.ShapeDtypeStruct((BATCH, DIM), jnp.float32),
    scratch_types=[
        pltpu.VMEM((BATCH + L,), jnp.int32),
        pltpu.VMEM((BATCH + L,), jnp.int32),
        pltpu.VMEM((BATCH + L,), jnp.int32),
        pltpu.VMEM((WINC, 64, 128), jnp.float32),
        pltpu.VMEM((WINC, 64, 128), jnp.float32),
        pltpu.VMEM((RING, DIM), jnp.float32),
        pltpu.SMEM((CMAX + 1,), jnp.int32),
        pltpu.SMEM((CMAX + 1,), jnp.int32),
        pltpu.SemaphoreType.DMA,
        pltpu.SemaphoreType.DMA,
        pltpu.SemaphoreType.DMA,
    ],
    compiler_params=pltpu.CompilerParams(needs_layout_passes=False),
)
def _gather_kernel(tabT_hbm, idx_hbm, out_hbm, sorted_v, hit_win, hit_val,
                   buf0, buf1, ring_v, starts_s, cur_s,
                   sem0, sem1, sem_out):
    wid = lax.axis_index("s") * NC + lax.axis_index("c")
    w0 = wid * WIN_LO + jnp.minimum(wid, WIN_EXTRA)
    count = WIN_LO + jnp.where(wid < WIN_EXTRA, 1, 0)

    pltpu.sync_copy(idx_hbm, sorted_v.at[pl.ds(0, BATCH)])

    lanes = lax.iota(jnp.int32, L)
    w1 = w0 + count

    def fetch_col(w, kc, buf, sem):
        tc = w * WINC + kc
        off = pl.multiple_of(jnp.where(tc < NTILECOL, tc, 0) << 7, 128)
        pltpu.make_async_copy(
            tabT_hbm.at[pl.ds(0, 64), pl.ds(off, 128)],
            buf.at[kc], sem,
        ).start()

    def fetch(w, buf, sem):
        for kc in range(WINC):
            fetch_col(w, kc, buf, sem)

    fetch(w0, buf0, sem0)

    @pl.when(count > 1)
    def _():
        fetch(w0 + 1, buf1, sem1)

    def prefilter(t, cur):
        v = sorted_v[pl.ds(t * L, L)]
        wv = v >> 9
        m = (wv >= w0) & (wv < w1)
        plsc.store_compressed(
            hit_win.at[pl.ds(cur, L)], (v >> 7) - 4 * w0, mask=m)
        plsc.store_compressed(
            hit_val.at[pl.ds(cur, L)],
            ((t * L + lanes) << 7) | (v & 127), mask=m)
        cnt = lax.squeeze(
            lax.slice(plsc.all_reduce_population_count(m), (0,), (1,)), (0,))
        return cur + cnt

    nh = lax.fori_loop(0, BATCH // L, prefilter, jnp.int32(0))
    hit_win[pl.ds(nh, L)] = jnp.full((L,), -1, jnp.int32)
    nvec = (nh + L - 1) >> 4

    def zero(k, _):
        cur_s[k] = 0
        return ()

    lax.fori_loop(0, CMAX + 1, zero, ())

    def count_pass(t, _):
        wv = hit_win[pl.ds(t * L, L)]
        for l in range(L):
            w = lax.squeeze(lax.slice(wv, (l,), (l + 1,)), (0,))

            @pl.when(w >= 0)
            def _():
                cur_s[w] = cur_s[w] + 1
        return ()

    lax.fori_loop(0, nvec, count_pass, ())

    starts_s[0] = 0

    def prefix(k, _):
        starts_s[k + 1] = starts_s[k] + cur_s[k]
        cur_s[k] = starts_s[k]
        return ()

    lax.fori_loop(0, CMAX, prefix, ())

    def place_pass(t, _):
        wv = hit_win[pl.ds(t * L, L)]
        vv = hit_val[pl.ds(t * L, L)]
        for l in range(L):
            w = lax.squeeze(lax.slice(wv, (l,), (l + 1,)), (0,))

            @pl.when(w >= 0)
            def _():
                d = cur_s[w]
                cur_s[w] = d + 1
                plsc.store_scatter(
                    sorted_v, [jnp.full((L,), d, jnp.int32)], vv,
                    mask=lanes == l)
        return ()

    lax.fori_loop(0, nvec, place_pass, ())

    bufs = (buf0, buf1)
    sems = (sem0, sem1)

    def wait_col(buf, kc, sem):
        pltpu.make_async_copy(
            tabT_hbm.at[pl.ds(0, 64), pl.ds(0, 128)], buf.at[kc], sem
        ).wait()

    def drain_one():
        pltpu.make_async_copy(
            out_hbm.at[pl.ds(0, 1)], ring_v.at[pl.ds(0, 1)], sem_out
        ).wait()

    def scan_col(k, kc, buf, cnt0):
        c0 = k * WINC + kc
        s0 = starts_s[c0]
        s1 = starts_s[c0 + 1]

        def per_hit(h, c):
            val = lax.squeeze(
                lax.slice(sorted_v[pl.ds(h, L)], (0,), (1,)), (0,))

            @pl.when(c >= RING)
            def _():
                drain_one()
            slot = c & (RING - 1)
            j = val >> 7
            kcol = jnp.full((L,), kc, jnp.int32)
            col = jnp.full((L,), val & 127, jnp.int32)
            for q in range(4):
                dvec = 16 * q + lanes
                vals = plsc.load_gather(buf, [kcol, dvec, col])
                ring_v[slot, pl.ds(16 * q, L)] = vals
            pltpu.make_async_copy(
                ring_v.at[pl.ds(slot, 1)],
                out_hbm.at[pl.ds(j, 1)], sem_out,
            ).start()
            return c + 1

        return lax.fori_loop(s0, s1, per_hit, cnt0)

    def outer(kk, cnt):
        for b in range(2):
            k = 2 * kk + b

            def step(cnt):
                cnt2 = cnt
                for kc in range(WINC):
                    c0 = k * WINC + kc
                    used = (k < 2) | (starts_s[c0 + 1] > starts_s[c0])

                    @pl.when(used)
                    def _():
                        wait_col(bufs[b], kc, sems[b])
                    cnt2 = lax.cond(
                        starts_s[c0 + 1] > starts_s[c0],
                        lambda c, kc=kc: scan_col(k, kc, bufs[b], c),
                        lambda c: c, cnt2)

                @pl.when(k + 2 < count)
                def _():
                    kf = k + 2
                    for kc in range(WINC):
                        cf = kf * WINC + kc

                        @pl.when(starts_s[cf + 1] > starts_s[cf])
                        def _(kc=kc):
                            fetch_col(w0 + kf, kc, bufs[b], sems[b])

                return cnt2

            cnt = lax.cond(k < count, step, lambda c: c, cnt)
        return cnt

    total = lax.fori_loop(0, (count + 1) >> 1, outer, jnp.int32(0))

    def drain(i, _):
        drain_one()
        return ()

    lax.fori_loop(0, jnp.minimum(total, RING), drain, ())


def kernel(inputs, table):
    return _gather_kernel(table.T, inputs)
